# Initial kernel scaffold; baseline (speedup 1.0000x reference)
import jax
import jax.numpy as jnp
from jax.experimental import pallas as pl


def kernel(x):
    raise NotImplementedError("write your pallas kernel here")


# speedup vs baseline: 1.5916x; 1.5916x over previous
import functools

import jax
import jax.numpy as jnp
import numpy as np
from jax import lax
from jax.experimental import pallas as pl
from jax.experimental.pallas import tpu as pltpu
from jax.experimental.pallas import tpu_sc as plsc

BH, BW = 32, 32

_GATHER_DNUMS = lax.GatherDimensionNumbers(
    offset_dims=(), collapsed_slice_dims=(0,), start_index_map=(0,))


def _dyn_gather(vec, idx):
    return lax.gather(vec, idx[:, None], _GATHER_DNUMS, slice_sizes=(1,),
                      mode=lax.GatherScatterMode.PROMISE_IN_BOUNDS)


def _perm_wtab(B, C, H, W):
    hb, wb = H // BH, W // BW
    n = hb * wb
    keys = jax.random.split(jax.random.key(42), B)
    perms = jnp.stack([jax.random.permutation(keys[i], n) for i in range(B)])
    si = perms // wb
    sj = perms % wb
    wtab = (si * (BH * wb) + sj).astype(jnp.int32).reshape(-1)
    pad = (B * n + 16 + 15) // 16 * 16 - B * n
    return jnp.concatenate([wtab, jnp.zeros((pad,), jnp.int32)])


def _make_sc_call(B, C, H, W, wtab_len):
    hb, wb = H // BH, W // BW
    n = hb * wb
    rows_total = B * C * H * W // BW
    strip = BH * wb
    region = hb * strip
    n_strips = B * C * hb
    NW = 32
    per_w = n_strips // NW
    n_grp = strip // 16
    f = np.arange(strip, dtype=np.int32)
    jv_np = (f % wb).reshape(n_grp, 16)
    hr12_np = ((f // wb) * wb).reshape(n_grp, 16)
    mesh = plsc.VectorSubcoreMesh(core_axis_name="c", subcore_axis_name="s")

    @functools.partial(
        pl.kernel,
        out_type=jax.ShapeDtypeStruct((rows_total, BW), jnp.float32),
        mesh=mesh,
        scratch_types=[
            pltpu.VMEM((wtab_len,), jnp.int32),
            pltpu.VMEM((3, 128), jnp.int32),
            pltpu.VMEM((strip, BW), jnp.float32),
            pltpu.SemaphoreType.DMA,
        ],
        compiler_params=pltpu.CompilerParams(use_tc_tiling_on_sc=False),
    )
    def sc_call(wtab_hbm, x_hbm, out_hbm, wtab_v, idx_v, rows_v, gsem):
        cid = lax.axis_index("c")
        sid = lax.axis_index("s")
        wid = sid * 2 + cid
        pltpu.sync_copy(wtab_hbm, wtab_v)

        def step(t, carry):
            g = wid * per_w + t
            b = g // (C * hb)
            i = lax.rem(g, hb)
            pb = b * n + i * wb
            base = (g // hb) * region
            wvec = wtab_v[pl.ds(pb, 16)]
            lanes = lax.iota(jnp.int32, 16)
            for gg in range(n_grp):
                fv = lanes + (gg * 16)
                jvv = lax.rem(fv, wb)
                w_g = _dyn_gather(wvec, jvv)
                idx_v[gg // 8, pl.ds((gg % 8) * 16, 16)] = (
                    w_g + (fv - jvv) + base)
            cps = [
                pltpu.async_copy(x_hbm.at[idx_v.at[k]],
                                 rows_v.at[pl.ds(k * 128, 128)], gsem)
                for k in range(3)
            ]
            for cp in cps:
                cp.wait()
            pltpu.sync_copy(rows_v, out_hbm.at[pl.ds(g * strip, strip)])
            return carry

        lax.fori_loop(0, per_w, step, 0)

    return sc_call


def kernel(x):
    B, C, H, W = x.shape
    wtab = _perm_wtab(B, C, H, W)
    xf = x.reshape(-1, BW)
    outf = _make_sc_call(B, C, H, W, wtab.shape[0])(wtab, xf)
    return outf.reshape(B, C, H, W)


# speedup vs baseline: 1.6806x; 1.0559x over previous
import functools

import jax
import jax.numpy as jnp
from jax import lax
from jax.experimental import pallas as pl
from jax.experimental.pallas import tpu as pltpu
from jax.experimental.pallas import tpu_sc as plsc

BH, BW = 32, 32
NBUF = 2

_GATHER_DNUMS = lax.GatherDimensionNumbers(
    offset_dims=(), collapsed_slice_dims=(0,), start_index_map=(0,))


def _dyn_gather(vec, idx):
    return lax.gather(vec, idx[:, None], _GATHER_DNUMS, slice_sizes=(1,),
                      mode=lax.GatherScatterMode.PROMISE_IN_BOUNDS)


def _perm_wtab(B, C, H, W):
    hb, wb = H // BH, W // BW
    n = hb * wb
    keys = jax.random.split(jax.random.key(42), B)
    perms = jnp.stack([jax.random.permutation(keys[i], n) for i in range(B)])
    si = perms // wb
    sj = perms % wb
    wtab = (si * (BH * wb) + sj).astype(jnp.int32).reshape(-1)
    pad = (B * n + 16 + 15) // 16 * 16 - B * n
    return jnp.concatenate([wtab, jnp.zeros((pad,), jnp.int32)])


def _make_sc_call(B, C, H, W, wtab_len):
    hb, wb = H // BH, W // BW
    n = hb * wb
    rows_total = B * C * H * W // BW
    strip = BH * wb
    region = hb * strip
    n_strips = B * C * hb
    NW = 32
    per_w = n_strips // NW
    n_grp = strip // 16
    n_tri = per_w // NBUF
    assert per_w % NBUF == 0
    mesh = plsc.VectorSubcoreMesh(core_axis_name="c", subcore_axis_name="s")

    @functools.partial(
        pl.kernel,
        out_type=jax.ShapeDtypeStruct((rows_total, BW), jnp.float32),
        mesh=mesh,
        scratch_types=[
            pltpu.VMEM((wtab_len,), jnp.int32),
            pltpu.VMEM((NBUF, 3, 128), jnp.int32),
            pltpu.VMEM((NBUF, strip, BW), jnp.float32),
            [pltpu.SemaphoreType.DMA] * NBUF,
            [pltpu.SemaphoreType.DMA] * NBUF,
        ],
        compiler_params=pltpu.CompilerParams(use_tc_tiling_on_sc=False),
    )
    def sc_call(wtab_hbm, x_hbm, out_hbm, wtab_v, idx_v, rows_v, gsems, ssems):
        cid = lax.axis_index("c")
        sid = lax.axis_index("s")
        wid = sid * 2 + cid
        g0 = wid * per_w
        pltpu.sync_copy(wtab_hbm, wtab_v)
        lanes = lax.iota(jnp.int32, 16)

        def run_gather(t, bi):
            g = g0 + t
            b = g // (C * hb)
            i = lax.rem(g, hb)
            pb = b * n + i * wb
            base = (g // hb) * region
            wvec = wtab_v[pl.ds(pb, 16)]
            for gg in range(n_grp):
                fv = lanes + (gg * 16)
                jvv = lax.rem(fv, wb)
                w_g = _dyn_gather(wvec, jvv)
                idx_v[bi, gg // 8, pl.ds((gg % 8) * 16, 16)] = (
                    w_g + (fv - jvv) + base)
            cps = [
                pltpu.async_copy(x_hbm.at[idx_v.at[bi, k]],
                                 rows_v.at[bi, pl.ds(k * 128, 128)], gsems[bi])
                for k in range(3)
            ]
            for cp in cps:
                cp.wait()

        def fire_scatter(t, bi):
            pltpu.async_copy(rows_v.at[bi],
                             out_hbm.at[pl.ds((g0 + t) * strip, strip)],
                             ssems[bi])

        def wait_scatter(t, bi):
            pltpu.make_async_copy(rows_v.at[bi],
                                  out_hbm.at[pl.ds((g0 + t) * strip, strip)],
                                  ssems[bi]).wait()

        def pair(u, carry):
            for bi in range(NBUF):
                t = u * NBUF + bi
                pl.when(u > 0)(lambda: wait_scatter(t - NBUF, bi))
                run_gather(t, bi)
                fire_scatter(t, bi)
            return carry

        lax.fori_loop(0, n_tri, pair, 0)
        wait_scatter(per_w - 2, 0)
        wait_scatter(per_w - 1, 1)

    return sc_call


def kernel(x):
    B, C, H, W = x.shape
    wtab = _perm_wtab(B, C, H, W)
    xf = x.reshape(-1, BW)
    outf = _make_sc_call(B, C, H, W, wtab.shape[0])(wtab, xf)
    return outf.reshape(B, C, H, W)
